# baseline (speedup 1.0000x reference)
import functools

import jax
import jax.numpy as jnp
from jax import lax
from jax.experimental import pallas as pl
from jax.experimental.pallas import tpu as pltpu
import jax.experimental.pallas.tpu_sc as plsc

TOPK = 2
TILE = 128

NC, NS, LANES = 2, 16, 16
NW = NC * NS


def _erf(z):
    s = jnp.sign(z)
    a = jnp.abs(z)
    t = 1.0 / (1.0 + 0.3275911 * a)
    poly = t * (0.254829592 + t * (-0.284496736 + t * (1.421413741
               + t * (-1.453152027 + t * 1.061405429))))
    return s * (1.0 - poly * jnp.exp(-a * a))


def _gelu(v):
    return 0.5 * v * (1.0 + _erf(v * 0.7071067811865476))


def _cumsum0(v):
    n = v.shape[0]
    k = 1
    while k < n:
        shifted = jnp.concatenate(
            [jnp.zeros((k, v.shape[1]), v.dtype), v[:-k]], axis=0)
        v = v + shifted
        k *= 2
    return v


def _gating_body(T, E, x_ref, gw_ref, gb_ref, pos_ref, wgt_ref, eid_ref):
    N = x_ref.shape[0]
    x = x_ref[...]
    logits = jnp.dot(x, gw_ref[...], preferred_element_type=jnp.float32)
    logits = logits + gb_ref[...]
    iota_e = lax.broadcasted_iota(jnp.int32, (N, E), 1)

    m1 = jnp.max(logits, axis=1, keepdims=True)
    i1 = jnp.min(jnp.where(logits == m1, iota_e, E), axis=1, keepdims=True)
    l2 = jnp.where(iota_e == i1, -jnp.inf, logits)
    m2 = jnp.max(l2, axis=1, keepdims=True)
    i2 = jnp.min(jnp.where(l2 == m2, iota_e, E), axis=1, keepdims=True)

    b = jnp.exp(m2 - m1)
    wa = 1.0 / (1.0 + b)
    wb = b / (1.0 + b)

    oh1 = (iota_e == i1).astype(jnp.float32)
    oh2 = (iota_e == i2).astype(jnp.float32)
    both = oh1 + oh2
    S = _cumsum0(both) - both
    counts = jnp.sum(both, axis=0, keepdims=True)
    pc = jnp.ceil(counts * (1.0 / TILE)) * TILE
    tri = (lax.broadcasted_iota(jnp.int32, (E, E), 0)
           < lax.broadcasted_iota(jnp.int32, (E, E), 1)).astype(jnp.float32)
    base = jnp.dot(pc, tri)

    pos1 = jnp.sum((base + S) * oh1, axis=1)
    pos2 = jnp.sum((base + S) * oh2, axis=1)
    pos_ref[0, :] = pos1.astype(jnp.int32)
    pos_ref[1, :] = pos2.astype(jnp.int32)
    wgt_ref[0, :, :] = jnp.broadcast_to(wa, (N, LANES))
    wgt_ref[1, :, :] = jnp.broadcast_to(wb, (N, LANES))

    ts = lax.broadcasted_iota(jnp.int32, (T, E), 0).astype(jnp.float32) \
        * float(TILE)
    cnt = jnp.sum((ts >= base).astype(jnp.int32), axis=1)
    eid_ref[0, :] = jnp.minimum(cnt - 1, E - 1)


def _gating(xf, gate_w, gate_b, T):
    N, D = xf.shape
    E = gate_w.shape[1]
    return pl.pallas_call(
        functools.partial(_gating_body, T, E),
        out_shape=(
            jax.ShapeDtypeStruct((TOPK, N), jnp.int32),
            jax.ShapeDtypeStruct((TOPK, N, LANES), jnp.float32),
            jax.ShapeDtypeStruct((1, T), jnp.int32),
        ),
    )(xf, gate_w, gate_b.reshape(1, E))


def _dispatch(xf, pos, P):
    N, D = xf.shape
    TPW = N // NW
    mesh = plsc.VectorSubcoreMesh(core_axis_name="c", subcore_axis_name="s")

    @functools.partial(
        pl.kernel,
        out_type=jax.ShapeDtypeStruct((P, D), jnp.float32),
        mesh=mesh,
        scratch_types=[
            pltpu.VMEM((TPW, D), jnp.float32),
            pltpu.VMEM((TPW,), jnp.int32),
            pltpu.VMEM((TPW,), jnp.int32),
            pltpu.SemaphoreType.DMA,
        ],
    )
    def run(x_hbm, pos_hbm, xs_hbm, rows_v, idx0_v, idx1_v, sem):
        wid = lax.axis_index("s") * NC + lax.axis_index("c")
        tok0 = wid * TPW
        pltpu.sync_copy(x_hbm.at[pl.ds(tok0, TPW)], rows_v)
        pltpu.sync_copy(pos_hbm.at[0, pl.ds(tok0, TPW)], idx0_v)
        pltpu.sync_copy(pos_hbm.at[1, pl.ds(tok0, TPW)], idx1_v)
        pltpu.async_copy(rows_v, xs_hbm.at[idx0_v], sem).wait()
        pltpu.async_copy(rows_v, xs_hbm.at[idx1_v], sem).wait()

    return run(xf, pos)


def _w_changed(eid_ref):
    t = pl.program_id(0)
    prev = eid_ref[jnp.maximum(t - 1, 0)]
    return (t == 0) | (eid_ref[t] != prev)


def _k1_body(eid_ref, xs_ref, w1_ref, b1_ref, h_ref, wbf_ref):
    @pl.when(_w_changed(eid_ref))
    def _():
        wbf_ref[...] = w1_ref[0].astype(jnp.bfloat16)

    acc = jnp.dot(xs_ref[...].astype(jnp.bfloat16), wbf_ref[...],
                  preferred_element_type=jnp.float32)
    h_ref[...] = _gelu(acc + b1_ref[0]).astype(jnp.bfloat16)


def _k1(tile_eid, xs, w1, b1, T):
    P, D = xs.shape
    E, _, H = w1.shape
    grid_spec = pltpu.PrefetchScalarGridSpec(
        num_scalar_prefetch=1,
        grid=(T,),
        in_specs=[
            pl.BlockSpec((TILE, D), lambda t, eid: (t, 0)),
            pl.BlockSpec((1, D, H), lambda t, eid: (eid[t], 0, 0)),
            pl.BlockSpec((1, 1, H), lambda t, eid: (eid[t], 0, 0)),
        ],
        out_specs=pl.BlockSpec((TILE, H), lambda t, eid: (t, 0)),
        scratch_shapes=[pltpu.VMEM((D, H), jnp.bfloat16)],
    )
    return pl.pallas_call(
        _k1_body,
        grid_spec=grid_spec,
        out_shape=jax.ShapeDtypeStruct((P, H), jnp.bfloat16),
        compiler_params=pltpu.CompilerParams(
            dimension_semantics=("arbitrary",)),
    )(tile_eid, xs, w1, b1.reshape(E, 1, H))


def _k2_body(eid_ref, h_ref, w2_ref, b2_ref, ys_ref, wbf_ref):
    @pl.when(_w_changed(eid_ref))
    def _():
        wbf_ref[...] = w2_ref[0].astype(jnp.bfloat16)

    acc = jnp.dot(h_ref[...], wbf_ref[...],
                  preferred_element_type=jnp.float32)
    ys_ref[...] = acc + b2_ref[0]


def _k2(tile_eid, h, w2, b2, T):
    P, H = h.shape
    E, _, D = w2.shape
    grid_spec = pltpu.PrefetchScalarGridSpec(
        num_scalar_prefetch=1,
        grid=(T,),
        in_specs=[
            pl.BlockSpec((TILE, H), lambda t, eid: (t, 0)),
            pl.BlockSpec((1, H, D), lambda t, eid: (eid[t], 0, 0)),
            pl.BlockSpec((1, 1, D), lambda t, eid: (eid[t], 0, 0)),
        ],
        out_specs=pl.BlockSpec((TILE, D), lambda t, eid: (t, 0)),
        scratch_shapes=[pltpu.VMEM((H, D), jnp.bfloat16)],
    )
    return pl.pallas_call(
        _k2_body,
        grid_spec=grid_spec,
        out_shape=jax.ShapeDtypeStruct((P, D), jnp.float32),
        compiler_params=pltpu.CompilerParams(
            dimension_semantics=("arbitrary",)),
    )(tile_eid, h, w2, b2.reshape(E, 1, D))


def _combine(ys, pos, wgt, N, D):
    TPW = N // NW
    CH = 32
    mesh = plsc.VectorSubcoreMesh(core_axis_name="c", subcore_axis_name="s")

    @functools.partial(
        pl.kernel,
        out_type=jax.ShapeDtypeStruct((N, D), jnp.float32),
        mesh=mesh,
        scratch_types=[
            pltpu.VMEM((CH, D), jnp.float32),
            pltpu.VMEM((CH, D), jnp.float32),
            pltpu.VMEM((CH,), jnp.int32),
            pltpu.VMEM((CH,), jnp.int32),
            pltpu.VMEM((CH, LANES), jnp.float32),
            pltpu.VMEM((CH, LANES), jnp.float32),
            pltpu.SemaphoreType.DMA,
        ],
    )
    def run(ys_hbm, pos_hbm, wgt_hbm, out_hbm,
            a_v, b_v, i0_v, i1_v, w0_v, w1_v, sem):
        wid = lax.axis_index("s") * NC + lax.axis_index("c")

        def chunk(j, carry):
            tok0 = wid * TPW + j * CH
            pltpu.sync_copy(pos_hbm.at[0, pl.ds(tok0, CH)], i0_v)
            pltpu.sync_copy(pos_hbm.at[1, pl.ds(tok0, CH)], i1_v)
            pltpu.sync_copy(wgt_hbm.at[0, pl.ds(tok0, CH)], w0_v)
            pltpu.sync_copy(wgt_hbm.at[1, pl.ds(tok0, CH)], w1_v)
            pltpu.async_copy(ys_hbm.at[i0_v], a_v, sem).wait()
            pltpu.async_copy(ys_hbm.at[i1_v], b_v, sem).wait()

            def row(r, c2):
                w0 = w0_v[r, :]
                w1 = w1_v[r, :]

                def col(cc, c3):
                    for u in range(4):
                        sl = pl.ds((cc * 4 + u) * LANES, LANES)
                        a_v[r, sl] = w0 * a_v[r, sl] + w1 * b_v[r, sl]
                    return c3
                return lax.fori_loop(0, D // (4 * LANES), col, c2)

            lax.fori_loop(0, CH, row, 0)
            pltpu.sync_copy(a_v, out_hbm.at[pl.ds(tok0, CH)])
            return carry

        lax.fori_loop(0, TPW // CH, chunk, 0)

    return run(ys, pos, wgt)


def kernel(x, gate_w, gate_b, w1, b1, w2, b2):
    B, L, D = x.shape
    N = B * L
    E = gate_w.shape[1]
    T = (N * TOPK) // TILE + (E - 1)
    P = T * TILE

    xf = x.reshape(N, D)
    pos, wgt, tile_eid = _gating(xf, gate_w, gate_b, T)
    tile_eid = tile_eid.reshape(T)
    xs = _dispatch(xf, pos, P)
    h = _k1(tile_eid, xs, w1, b1, T)
    ys = _k2(tile_eid, h, w2, b2, T)
    out = _combine(ys, pos, wgt, N, D)
    return out.reshape(B, L, D)


# speedup vs baseline: 1.2284x; 1.2284x over previous
import functools

import jax
import jax.numpy as jnp
from jax import lax
from jax.experimental import pallas as pl
from jax.experimental.pallas import tpu as pltpu
import jax.experimental.pallas.tpu_sc as plsc

TOPK = 2
TILE = 128

NC, NS, LANES = 2, 16, 16
NW = NC * NS


def _gelu(v):
    return 0.5 * v * (1.0 + lax.erf(v * 0.7071067811865476))


def _cumsum0(v):
    n = v.shape[0]
    k = 1
    while k < n:
        shifted = jnp.concatenate(
            [jnp.zeros((k, v.shape[1]), v.dtype), v[:-k]], axis=0)
        v = v + shifted
        k *= 2
    return v


def _gating_body(T, E, x_ref, gw_ref, gb_ref, pos_ref, wgt_ref, eid_ref):
    N = x_ref.shape[0]
    x = x_ref[...]
    logits = jnp.dot(x, gw_ref[...], preferred_element_type=jnp.float32)
    logits = logits + gb_ref[...]
    iota_e = lax.broadcasted_iota(jnp.int32, (N, E), 1)

    m1 = jnp.max(logits, axis=1, keepdims=True)
    i1 = jnp.min(jnp.where(logits == m1, iota_e, E), axis=1, keepdims=True)
    l2 = jnp.where(iota_e == i1, -jnp.inf, logits)
    m2 = jnp.max(l2, axis=1, keepdims=True)
    i2 = jnp.min(jnp.where(l2 == m2, iota_e, E), axis=1, keepdims=True)

    b = jnp.exp(m2 - m1)
    wa = 1.0 / (1.0 + b)
    wb = b / (1.0 + b)

    oh1 = (iota_e == i1).astype(jnp.float32)
    oh2 = (iota_e == i2).astype(jnp.float32)
    both = oh1 + oh2
    S = _cumsum0(both) - both
    counts = jnp.sum(both, axis=0, keepdims=True)
    pc = jnp.ceil(counts * (1.0 / TILE)) * TILE
    tri = (lax.broadcasted_iota(jnp.int32, (E, E), 0)
           < lax.broadcasted_iota(jnp.int32, (E, E), 1)).astype(jnp.float32)
    base = jnp.dot(pc, tri)

    pos1 = jnp.sum((base + S) * oh1, axis=1)
    pos2 = jnp.sum((base + S) * oh2, axis=1)
    pos_ref[0, :] = pos1.astype(jnp.int32)
    pos_ref[1, :] = pos2.astype(jnp.int32)
    wgt_ref[0, :, :] = jnp.broadcast_to(wa, (N, LANES))
    wgt_ref[1, :, :] = jnp.broadcast_to(wb, (N, LANES))

    ts = lax.broadcasted_iota(jnp.int32, (T, E), 0).astype(jnp.float32) \
        * float(TILE)
    cnt = jnp.sum((ts >= base).astype(jnp.int32), axis=1)
    eid_ref[0, :] = jnp.minimum(cnt - 1, E - 1)


def _gating(xf, gate_w, gate_b, T):
    N, D = xf.shape
    E = gate_w.shape[1]
    return pl.pallas_call(
        functools.partial(_gating_body, T, E),
        out_shape=(
            jax.ShapeDtypeStruct((TOPK, N), jnp.int32),
            jax.ShapeDtypeStruct((TOPK, N, LANES), jnp.float32),
            jax.ShapeDtypeStruct((1, T), jnp.int32),
        ),
    )(xf, gate_w, gate_b.reshape(1, E))


def _dispatch(xf, pos, P):
    N, D = xf.shape
    TPW = N // NW
    mesh = plsc.VectorSubcoreMesh(core_axis_name="c", subcore_axis_name="s")

    @functools.partial(
        pl.kernel,
        out_type=jax.ShapeDtypeStruct((P, D), jnp.float32),
        mesh=mesh,
        scratch_types=[
            pltpu.VMEM((TPW, D), jnp.float32),
            pltpu.VMEM((TPW,), jnp.int32),
            pltpu.VMEM((TPW,), jnp.int32),
            pltpu.SemaphoreType.DMA,
        ],
    )
    def run(x_hbm, pos_hbm, xs_hbm, rows_v, idx0_v, idx1_v, sem):
        wid = lax.axis_index("s") * NC + lax.axis_index("c")
        tok0 = wid * TPW
        pltpu.sync_copy(x_hbm.at[pl.ds(tok0, TPW)], rows_v)
        pltpu.sync_copy(pos_hbm.at[0, pl.ds(tok0, TPW)], idx0_v)
        pltpu.sync_copy(pos_hbm.at[1, pl.ds(tok0, TPW)], idx1_v)
        pltpu.async_copy(rows_v, xs_hbm.at[idx0_v], sem).wait()
        pltpu.async_copy(rows_v, xs_hbm.at[idx1_v], sem).wait()

    return run(xf, pos)


def _w_changed(eid_ref):
    t = pl.program_id(0)
    prev = eid_ref[jnp.maximum(t - 1, 0)]
    return (t == 0) | (eid_ref[t] != prev)


def _k1_body(eid_ref, xs_ref, w1_ref, b1_ref, h_ref, wbf_ref):
    @pl.when(_w_changed(eid_ref))
    def _():
        wbf_ref[...] = w1_ref[0].astype(jnp.bfloat16)

    acc = jnp.dot(xs_ref[...].astype(jnp.bfloat16), wbf_ref[...],
                  preferred_element_type=jnp.float32)
    h_ref[...] = _gelu(acc + b1_ref[0]).astype(jnp.bfloat16)


def _k1(tile_eid, xs, w1, b1, T):
    P, D = xs.shape
    E, _, H = w1.shape
    grid_spec = pltpu.PrefetchScalarGridSpec(
        num_scalar_prefetch=1,
        grid=(T,),
        in_specs=[
            pl.BlockSpec((TILE, D), lambda t, eid: (t, 0)),
            pl.BlockSpec((1, D, H), lambda t, eid: (eid[t], 0, 0)),
            pl.BlockSpec((1, 1, H), lambda t, eid: (eid[t], 0, 0)),
        ],
        out_specs=pl.BlockSpec((TILE, H), lambda t, eid: (t, 0)),
        scratch_shapes=[pltpu.VMEM((D, H), jnp.bfloat16)],
    )
    return pl.pallas_call(
        _k1_body,
        grid_spec=grid_spec,
        out_shape=jax.ShapeDtypeStruct((P, H), jnp.bfloat16),
        compiler_params=pltpu.CompilerParams(
            dimension_semantics=("arbitrary",)),
    )(tile_eid, xs, w1, b1.reshape(E, 1, H))


def _k2_body(eid_ref, h_ref, w2_ref, b2_ref, ys_ref, wbf_ref):
    @pl.when(_w_changed(eid_ref))
    def _():
        wbf_ref[...] = w2_ref[0].astype(jnp.bfloat16)

    acc = jnp.dot(h_ref[...], wbf_ref[...],
                  preferred_element_type=jnp.float32)
    ys_ref[...] = acc + b2_ref[0]


def _k2(tile_eid, h, w2, b2, T):
    P, H = h.shape
    E, _, D = w2.shape
    grid_spec = pltpu.PrefetchScalarGridSpec(
        num_scalar_prefetch=1,
        grid=(T,),
        in_specs=[
            pl.BlockSpec((TILE, H), lambda t, eid: (t, 0)),
            pl.BlockSpec((1, H, D), lambda t, eid: (eid[t], 0, 0)),
            pl.BlockSpec((1, 1, D), lambda t, eid: (eid[t], 0, 0)),
        ],
        out_specs=pl.BlockSpec((TILE, D), lambda t, eid: (t, 0)),
        scratch_shapes=[pltpu.VMEM((H, D), jnp.bfloat16)],
    )
    return pl.pallas_call(
        _k2_body,
        grid_spec=grid_spec,
        out_shape=jax.ShapeDtypeStruct((P, D), jnp.float32),
        compiler_params=pltpu.CompilerParams(
            dimension_semantics=("arbitrary",)),
    )(tile_eid, h, w2, b2.reshape(E, 1, D))


def _combine(ys, pos, wgt, N, D):
    TPW = N // NW
    CH = 32
    mesh = plsc.VectorSubcoreMesh(core_axis_name="c", subcore_axis_name="s")

    @functools.partial(
        pl.kernel,
        out_type=jax.ShapeDtypeStruct((N, D), jnp.float32),
        mesh=mesh,
        scratch_types=[
            pltpu.VMEM((CH, D), jnp.float32),
            pltpu.VMEM((CH, D), jnp.float32),
            pltpu.VMEM((CH,), jnp.int32),
            pltpu.VMEM((CH,), jnp.int32),
            pltpu.VMEM((CH, LANES), jnp.float32),
            pltpu.VMEM((CH, LANES), jnp.float32),
            pltpu.SemaphoreType.DMA,
            pltpu.SemaphoreType.DMA,
        ],
    )
    def run(ys_hbm, pos_hbm, wgt_hbm, out_hbm,
            a_v, b_v, i0_v, i1_v, w0_v, w1_v, sem_a, sem_b):
        wid = lax.axis_index("s") * NC + lax.axis_index("c")

        for j in range(TPW // CH):
            tok0 = wid * TPW + j * CH
            pltpu.sync_copy(pos_hbm.at[0, pl.ds(tok0, CH)], i0_v)
            pltpu.sync_copy(pos_hbm.at[1, pl.ds(tok0, CH)], i1_v)
            pltpu.sync_copy(wgt_hbm.at[0, pl.ds(tok0, CH)], w0_v)
            pltpu.sync_copy(wgt_hbm.at[1, pl.ds(tok0, CH)], w1_v)
            cp_a = pltpu.async_copy(ys_hbm.at[i0_v], a_v, sem_a)
            cp_b = pltpu.async_copy(ys_hbm.at[i1_v], b_v, sem_b)
            cp_a.wait()
            cp_b.wait()

            for r in range(CH):
                w0 = w0_v[r, :]
                w1 = w1_v[r, :]

                def col(cc, c3, _r=r, _w0=w0, _w1=w1):
                    for u in range(4):
                        sl = pl.ds((cc * 4 + u) * LANES, LANES)
                        a_v[_r, sl] = _w0 * a_v[_r, sl] + _w1 * b_v[_r, sl]
                    return c3
                lax.fori_loop(0, D // (4 * LANES), col, 0)

            pltpu.sync_copy(a_v, out_hbm.at[pl.ds(tok0, CH)])

    return run(ys, pos, wgt)


def kernel(x, gate_w, gate_b, w1, b1, w2, b2):
    B, L, D = x.shape
    N = B * L
    E = gate_w.shape[1]
    T = (N * TOPK) // TILE + (E - 1)
    P = T * TILE

    xf = x.reshape(N, D)
    pos, wgt, tile_eid = _gating(xf, gate_w, gate_b, T)
    tile_eid = tile_eid.reshape(T)
    xs = _dispatch(xf, pos, P)
    h = _k1(tile_eid, xs, w1, b1, T)
    ys = _k2(tile_eid, h, w2, b2, T)
    out = _combine(ys, pos, wgt, N, D)
    return out.reshape(B, L, D)


# speedup vs baseline: 1.2393x; 1.0089x over previous
import functools

import jax
import jax.numpy as jnp
from jax import lax
from jax.experimental import pallas as pl
from jax.experimental.pallas import tpu as pltpu
import jax.experimental.pallas.tpu_sc as plsc

TOPK = 2
TILE = 128

NC, NS, LANES = 2, 16, 16
NW = NC * NS


def _gelu(v):
    return 0.5 * v * (1.0 + lax.erf(v * 0.7071067811865476))


def _cumsum0(v):
    n = v.shape[0]
    k = 1
    while k < n:
        shifted = jnp.concatenate(
            [jnp.zeros((k, v.shape[1]), v.dtype), v[:-k]], axis=0)
        v = v + shifted
        k *= 2
    return v


def _gating_body(T, E, x_ref, gw_ref, gb_ref, pos_ref, wgt_ref, eid_ref):
    N = x_ref.shape[0]
    x = x_ref[...]
    logits = jnp.dot(x, gw_ref[...], preferred_element_type=jnp.float32)
    logits = logits + gb_ref[...]
    iota_e = lax.broadcasted_iota(jnp.int32, (N, E), 1)

    m1 = jnp.max(logits, axis=1, keepdims=True)
    i1 = jnp.min(jnp.where(logits == m1, iota_e, E), axis=1, keepdims=True)
    l2 = jnp.where(iota_e == i1, -jnp.inf, logits)
    m2 = jnp.max(l2, axis=1, keepdims=True)
    i2 = jnp.min(jnp.where(l2 == m2, iota_e, E), axis=1, keepdims=True)

    b = jnp.exp(m2 - m1)
    wa = 1.0 / (1.0 + b)
    wb = b / (1.0 + b)

    oh1 = (iota_e == i1).astype(jnp.float32)
    oh2 = (iota_e == i2).astype(jnp.float32)
    both = oh1 + oh2
    S = _cumsum0(both) - both
    counts = jnp.sum(both, axis=0, keepdims=True)
    pc = jnp.ceil(counts * (1.0 / TILE)) * TILE
    tri = (lax.broadcasted_iota(jnp.int32, (E, E), 0)
           < lax.broadcasted_iota(jnp.int32, (E, E), 1)).astype(jnp.float32)
    base = jnp.dot(pc, tri)

    pos1 = jnp.sum((base + S) * oh1, axis=1)
    pos2 = jnp.sum((base + S) * oh2, axis=1)
    pos_ref[0, :] = pos1.astype(jnp.int32)
    pos_ref[1, :] = pos2.astype(jnp.int32)
    wgt_ref[0, :, :] = jnp.broadcast_to(wa, (N, LANES))
    wgt_ref[1, :, :] = jnp.broadcast_to(wb, (N, LANES))

    ts = lax.broadcasted_iota(jnp.int32, (T, E), 0).astype(jnp.float32) \
        * float(TILE)
    cnt = jnp.sum((ts >= base).astype(jnp.int32), axis=1)
    eid_ref[0, :] = jnp.minimum(cnt - 1, E - 1)


def _gating(xf, gate_w, gate_b, T):
    N, D = xf.shape
    E = gate_w.shape[1]
    return pl.pallas_call(
        functools.partial(_gating_body, T, E),
        out_shape=(
            jax.ShapeDtypeStruct((TOPK, N), jnp.int32),
            jax.ShapeDtypeStruct((TOPK, N, LANES), jnp.float32),
            jax.ShapeDtypeStruct((1, T), jnp.int32),
        ),
    )(xf, gate_w, gate_b.reshape(1, E))


def _dispatch(xf, pos, P):
    N, D = xf.shape
    TPW = N // NW
    mesh = plsc.VectorSubcoreMesh(core_axis_name="c", subcore_axis_name="s")

    @functools.partial(
        pl.kernel,
        out_type=jax.ShapeDtypeStruct((P, D), jnp.float32),
        mesh=mesh,
        scratch_types=[
            pltpu.VMEM((TPW, D), jnp.float32),
            pltpu.VMEM((TPW,), jnp.int32),
            pltpu.VMEM((TPW,), jnp.int32),
            pltpu.SemaphoreType.DMA,
        ],
    )
    def run(x_hbm, pos_hbm, xs_hbm, rows_v, idx0_v, idx1_v, sem):
        wid = lax.axis_index("s") * NC + lax.axis_index("c")
        tok0 = wid * TPW
        pltpu.sync_copy(x_hbm.at[pl.ds(tok0, TPW)], rows_v)
        pltpu.sync_copy(pos_hbm.at[0, pl.ds(tok0, TPW)], idx0_v)
        pltpu.sync_copy(pos_hbm.at[1, pl.ds(tok0, TPW)], idx1_v)
        pltpu.async_copy(rows_v, xs_hbm.at[idx0_v], sem).wait()
        pltpu.async_copy(rows_v, xs_hbm.at[idx1_v], sem).wait()

    return run(xf, pos)


def _w_changed(eid_ref):
    t = pl.program_id(0)
    prev = eid_ref[jnp.maximum(t - 1, 0)]
    return (t == 0) | (eid_ref[t] != prev)


def _k1_body(eid_ref, xs_ref, w1_ref, b1_ref, h_ref, wbf_ref):
    @pl.when(_w_changed(eid_ref))
    def _():
        wbf_ref[...] = w1_ref[0].astype(jnp.bfloat16)

    acc = jnp.dot(xs_ref[...].astype(jnp.bfloat16), wbf_ref[...],
                  preferred_element_type=jnp.float32)
    h_ref[...] = _gelu(acc + b1_ref[0]).astype(jnp.bfloat16)


def _k1(tile_eid, xs, w1, b1, T):
    P, D = xs.shape
    E, _, H = w1.shape
    grid_spec = pltpu.PrefetchScalarGridSpec(
        num_scalar_prefetch=1,
        grid=(T,),
        in_specs=[
            pl.BlockSpec((TILE, D), lambda t, eid: (t, 0)),
            pl.BlockSpec((1, D, H), lambda t, eid: (eid[t], 0, 0)),
            pl.BlockSpec((1, 1, H), lambda t, eid: (eid[t], 0, 0)),
        ],
        out_specs=pl.BlockSpec((TILE, H), lambda t, eid: (t, 0)),
        scratch_shapes=[pltpu.VMEM((D, H), jnp.bfloat16)],
    )
    return pl.pallas_call(
        _k1_body,
        grid_spec=grid_spec,
        out_shape=jax.ShapeDtypeStruct((P, H), jnp.bfloat16),
        compiler_params=pltpu.CompilerParams(
            dimension_semantics=("arbitrary",)),
    )(tile_eid, xs, w1, b1.reshape(E, 1, H))


def _k2_body(eid_ref, h_ref, w2_ref, b2_ref, ys_ref, wbf_ref):
    @pl.when(_w_changed(eid_ref))
    def _():
        wbf_ref[...] = w2_ref[0].astype(jnp.bfloat16)

    acc = jnp.dot(h_ref[...], wbf_ref[...],
                  preferred_element_type=jnp.float32)
    ys_ref[...] = acc + b2_ref[0]


def _k2(tile_eid, h, w2, b2, T):
    P, H = h.shape
    E, _, D = w2.shape
    grid_spec = pltpu.PrefetchScalarGridSpec(
        num_scalar_prefetch=1,
        grid=(T,),
        in_specs=[
            pl.BlockSpec((TILE, H), lambda t, eid: (t, 0)),
            pl.BlockSpec((1, H, D), lambda t, eid: (eid[t], 0, 0)),
            pl.BlockSpec((1, 1, D), lambda t, eid: (eid[t], 0, 0)),
        ],
        out_specs=pl.BlockSpec((TILE, D), lambda t, eid: (t, 0)),
        scratch_shapes=[pltpu.VMEM((H, D), jnp.bfloat16)],
    )
    return pl.pallas_call(
        _k2_body,
        grid_spec=grid_spec,
        out_shape=jax.ShapeDtypeStruct((P, D), jnp.float32),
        compiler_params=pltpu.CompilerParams(
            dimension_semantics=("arbitrary",)),
    )(tile_eid, h, w2, b2.reshape(E, 1, D))


def _combine(ys, pos, wgt, N, D):
    TPW = N // NW
    CH = 16
    NCHUNK = TPW // CH
    mesh = plsc.VectorSubcoreMesh(core_axis_name="c", subcore_axis_name="s")

    @functools.partial(
        pl.kernel,
        out_type=jax.ShapeDtypeStruct((N, D), jnp.float32),
        mesh=mesh,
        scratch_types=[
            pltpu.VMEM((2, CH, D), jnp.float32),
            pltpu.VMEM((2, CH, D), jnp.float32),
            pltpu.VMEM((2, CH), jnp.int32),
            pltpu.VMEM((2, CH), jnp.int32),
            pltpu.VMEM((2, CH, LANES), jnp.float32),
            pltpu.VMEM((2, CH, LANES), jnp.float32),
            pltpu.SemaphoreType.DMA,
            pltpu.SemaphoreType.DMA,
            pltpu.SemaphoreType.DMA,
            pltpu.SemaphoreType.DMA,
        ],
    )
    def run(ys_hbm, pos_hbm, wgt_hbm, out_hbm,
            a_v, b_v, i0_v, i1_v, w0_v, w1_v, s_a0, s_b0, s_a1, s_b1):
        wid = lax.axis_index("s") * NC + lax.axis_index("c")
        sems = ((s_a0, s_b0), (s_a1, s_b1))

        def issue(j):
            p = j % 2
            tok0 = wid * TPW + j * CH
            pltpu.sync_copy(pos_hbm.at[0, pl.ds(tok0, CH)], i0_v.at[p])
            pltpu.sync_copy(pos_hbm.at[1, pl.ds(tok0, CH)], i1_v.at[p])
            pltpu.sync_copy(wgt_hbm.at[0, pl.ds(tok0, CH)], w0_v.at[p])
            pltpu.sync_copy(wgt_hbm.at[1, pl.ds(tok0, CH)], w1_v.at[p])
            return (pltpu.async_copy(ys_hbm.at[i0_v.at[p]], a_v.at[p],
                                     sems[p][0]),
                    pltpu.async_copy(ys_hbm.at[i1_v.at[p]], b_v.at[p],
                                     sems[p][1]))

        pend = issue(0)
        for j in range(NCHUNK):
            p = j % 2
            nxt = issue(j + 1) if j + 1 < NCHUNK else None
            pend[0].wait()
            pend[1].wait()

            for r in range(CH):
                w0 = w0_v[p, r, :]
                w1 = w1_v[p, r, :]

                def col(cc, c3, _p=p, _r=r, _w0=w0, _w1=w1):
                    for u in range(4):
                        sl = pl.ds((cc * 4 + u) * LANES, LANES)
                        a_v[_p, _r, sl] = (_w0 * a_v[_p, _r, sl]
                                           + _w1 * b_v[_p, _r, sl])
                    return c3
                lax.fori_loop(0, D // (4 * LANES), col, 0)

            tok0 = wid * TPW + j * CH
            pltpu.sync_copy(a_v.at[p], out_hbm.at[pl.ds(tok0, CH)])
            pend = nxt

    return run(ys, pos, wgt)


def kernel(x, gate_w, gate_b, w1, b1, w2, b2):
    B, L, D = x.shape
    N = B * L
    E = gate_w.shape[1]
    T = (N * TOPK) // TILE + (E - 1)
    P = T * TILE

    xf = x.reshape(N, D)
    pos, wgt, tile_eid = _gating(xf, gate_w, gate_b, T)
    tile_eid = tile_eid.reshape(T)
    xs = _dispatch(xf, pos, P)
    h = _k1(tile_eid, xs, w1, b1, T)
    ys = _k2(tile_eid, h, w2, b2, T)
    out = _combine(ys, pos, wgt, N, D)
    return out.reshape(B, L, D)


# speedup vs baseline: 1.3370x; 1.0788x over previous
import functools

import jax
import jax.numpy as jnp
from jax import lax
from jax.experimental import pallas as pl
from jax.experimental.pallas import tpu as pltpu
import jax.experimental.pallas.tpu_sc as plsc

TOPK = 2
TILE = 128

NC, NS, LANES = 2, 16, 16
NW = NC * NS


def _gelu(v):
    return 0.5 * v * (1.0 + lax.erf(v * 0.7071067811865476))


def _cumsum0(v):
    n = v.shape[0]
    k = 1
    while k < n:
        shifted = jnp.concatenate(
            [jnp.zeros((k, v.shape[1]), v.dtype), v[:-k]], axis=0)
        v = v + shifted
        k *= 2
    return v


def _gating_body(T, E, x_ref, gw_ref, gb_ref, pos_ref, wgt_ref, eid_ref):
    N = x_ref.shape[0]
    x = x_ref[...]
    logits = jnp.dot(x, gw_ref[...], preferred_element_type=jnp.float32)
    logits = logits + gb_ref[...]
    iota_e = lax.broadcasted_iota(jnp.int32, (N, E), 1)

    m1 = jnp.max(logits, axis=1, keepdims=True)
    i1 = jnp.min(jnp.where(logits == m1, iota_e, E), axis=1, keepdims=True)
    l2 = jnp.where(iota_e == i1, -jnp.inf, logits)
    m2 = jnp.max(l2, axis=1, keepdims=True)
    i2 = jnp.min(jnp.where(l2 == m2, iota_e, E), axis=1, keepdims=True)

    b = jnp.exp(m2 - m1)
    wa = 1.0 / (1.0 + b)
    wb = b / (1.0 + b)

    oh1 = (iota_e == i1).astype(jnp.float32)
    oh2 = (iota_e == i2).astype(jnp.float32)
    both = oh1 + oh2
    S = _cumsum0(both) - both
    counts = jnp.sum(both, axis=0, keepdims=True)
    pc = jnp.ceil(counts * (1.0 / TILE)) * TILE
    tri = (lax.broadcasted_iota(jnp.int32, (E, E), 0)
           < lax.broadcasted_iota(jnp.int32, (E, E), 1)).astype(jnp.float32)
    base = jnp.dot(pc, tri)

    pos1 = jnp.sum((base + S) * oh1, axis=1)
    pos2 = jnp.sum((base + S) * oh2, axis=1)
    pos_ref[0, :] = pos1.astype(jnp.int32)
    pos_ref[1, :] = pos2.astype(jnp.int32)
    wgt_ref[0, :, :] = jnp.broadcast_to(wa, (N, LANES))
    wgt_ref[1, :, :] = jnp.broadcast_to(wb, (N, LANES))

    ts = lax.broadcasted_iota(jnp.int32, (T, E), 0).astype(jnp.float32) \
        * float(TILE)
    cnt = jnp.sum((ts >= base).astype(jnp.int32), axis=1)
    eid_ref[0, :] = jnp.minimum(cnt - 1, E - 1)


def _gating(xf, gate_w, gate_b, T):
    N, D = xf.shape
    E = gate_w.shape[1]
    return pl.pallas_call(
        functools.partial(_gating_body, T, E),
        out_shape=(
            jax.ShapeDtypeStruct((TOPK, N), jnp.int32),
            jax.ShapeDtypeStruct((TOPK, N, LANES), jnp.float32),
            jax.ShapeDtypeStruct((1, T), jnp.int32),
        ),
    )(xf, gate_w, gate_b.reshape(1, E))


def _dispatch(xf, pos, P):
    N, D = xf.shape
    TPW = N // NW
    mesh = plsc.VectorSubcoreMesh(core_axis_name="c", subcore_axis_name="s")

    @functools.partial(
        pl.kernel,
        out_type=jax.ShapeDtypeStruct((P, D), jnp.float32),
        mesh=mesh,
        scratch_types=[
            pltpu.VMEM((TPW, D), jnp.float32),
            pltpu.VMEM((TPW,), jnp.int32),
            pltpu.VMEM((TPW,), jnp.int32),
            pltpu.SemaphoreType.DMA,
        ],
    )
    def run(x_hbm, pos_hbm, xs_hbm, rows_v, idx0_v, idx1_v, sem):
        wid = lax.axis_index("s") * NC + lax.axis_index("c")
        tok0 = wid * TPW
        pltpu.sync_copy(x_hbm.at[pl.ds(tok0, TPW)], rows_v)
        pltpu.sync_copy(pos_hbm.at[0, pl.ds(tok0, TPW)], idx0_v)
        pltpu.sync_copy(pos_hbm.at[1, pl.ds(tok0, TPW)], idx1_v)
        pltpu.async_copy(rows_v, xs_hbm.at[idx0_v], sem).wait()
        pltpu.async_copy(rows_v, xs_hbm.at[idx1_v], sem).wait()

    return run(xf, pos)


def _run_info(tile_eid, E):
    change = jnp.concatenate(
        [jnp.zeros((1,), jnp.int32),
         (tile_eid[1:] != tile_eid[:-1]).astype(jnp.int32)])
    run_id = jnp.cumsum(change)
    nruns = (run_id[-1] + 1).reshape(1)
    run_expert = jnp.zeros((E,), jnp.int32).at[run_id].set(tile_eid)
    return run_id, run_expert, nruns


def _gmm_body(out_fn, cast_lhs, eid_ref, rid_ref, rexp_ref, nr_ref,
              lhs_ref, w_any, b_ref, o_ref, stag_ref, wbf_ref, s0, s1):
    t = pl.program_id(0)
    r = rid_ref[t]
    changed = (t == 0) | (rid_ref[jnp.maximum(t - 1, 0)] != r)
    R = nr_ref[0]
    sems = (s0, s1)

    @pl.when(t == 0)
    def _():
        pltpu.make_async_copy(
            w_any.at[rexp_ref[0]], stag_ref.at[0], s0).start()

        @pl.when(R > 1)
        def _():
            pltpu.make_async_copy(
                w_any.at[rexp_ref[1]], stag_ref.at[1], s1).start()

    for p in (0, 1):
        @pl.when(changed & (r % 2 == p))
        def _(p=p):
            pltpu.make_async_copy(
                w_any.at[rexp_ref[r]], stag_ref.at[p], sems[p]).wait()
            wbf_ref[...] = stag_ref[p].astype(jnp.bfloat16)

            @pl.when(r + 2 < R)
            def _():
                pltpu.make_async_copy(
                    w_any.at[rexp_ref[jnp.minimum(r + 2, rexp_ref.shape[0]
                                                  - 1)]],
                    stag_ref.at[p], sems[p]).start()

    lhs = lhs_ref[...]
    if cast_lhs:
        lhs = lhs.astype(jnp.bfloat16)
    acc = jnp.dot(lhs, wbf_ref[...], preferred_element_type=jnp.float32)
    o_ref[...] = out_fn(acc + b_ref[0])


def _gmm(tile_eid, lhs, w, b, T, out_dtype, out_fn, cast_lhs):
    P = lhs.shape[0]
    E, K, M = w.shape
    run_id, run_expert, nruns = _run_info(tile_eid, E)
    grid_spec = pltpu.PrefetchScalarGridSpec(
        num_scalar_prefetch=4,
        grid=(T,),
        in_specs=[
            pl.BlockSpec((TILE, K), lambda t, eid, rid, rexp, nr: (t, 0)),
            pl.BlockSpec(memory_space=pl.ANY),
            pl.BlockSpec((1, 1, M),
                         lambda t, eid, rid, rexp, nr: (eid[t], 0, 0)),
        ],
        out_specs=pl.BlockSpec((TILE, M), lambda t, eid, rid, rexp, nr:
                               (t, 0)),
        scratch_shapes=[
            pltpu.VMEM((2, K, M), jnp.float32),
            pltpu.VMEM((K, M), jnp.bfloat16),
            pltpu.SemaphoreType.DMA,
            pltpu.SemaphoreType.DMA,
        ],
    )
    return pl.pallas_call(
        functools.partial(_gmm_body, out_fn, cast_lhs),
        grid_spec=grid_spec,
        out_shape=jax.ShapeDtypeStruct((P, M), out_dtype),
        compiler_params=pltpu.CompilerParams(
            dimension_semantics=("arbitrary",)),
    )(tile_eid, run_id, run_expert, nruns, lhs, w, b.reshape(E, 1, M))


def _k1(tile_eid, xs, w1, b1, T):
    return _gmm(tile_eid, xs, w1, b1, T, jnp.bfloat16,
                lambda v: _gelu(v).astype(jnp.bfloat16), cast_lhs=True)


def _k2(tile_eid, h, w2, b2, T):
    return _gmm(tile_eid, h, w2, b2, T, jnp.float32,
                lambda v: v, cast_lhs=False)


def _combine(ys, pos, wgt, N, D):
    TPW = N // NW
    CH = 16
    NCHUNK = TPW // CH
    mesh = plsc.VectorSubcoreMesh(core_axis_name="c", subcore_axis_name="s")

    @functools.partial(
        pl.kernel,
        out_type=jax.ShapeDtypeStruct((N, D), jnp.float32),
        mesh=mesh,
        scratch_types=[
            pltpu.VMEM((2, CH, D), jnp.float32),
            pltpu.VMEM((2, CH, D), jnp.float32),
            pltpu.VMEM((2, CH), jnp.int32),
            pltpu.VMEM((2, CH), jnp.int32),
            pltpu.VMEM((2, CH, LANES), jnp.float32),
            pltpu.VMEM((2, CH, LANES), jnp.float32),
            pltpu.SemaphoreType.DMA,
            pltpu.SemaphoreType.DMA,
            pltpu.SemaphoreType.DMA,
            pltpu.SemaphoreType.DMA,
        ],
    )
    def run(ys_hbm, pos_hbm, wgt_hbm, out_hbm,
            a_v, b_v, i0_v, i1_v, w0_v, w1_v, s_a0, s_b0, s_a1, s_b1):
        wid = lax.axis_index("s") * NC + lax.axis_index("c")
        sems = ((s_a0, s_b0), (s_a1, s_b1))

        def issue(j):
            p = j % 2
            tok0 = wid * TPW + j * CH
            pltpu.sync_copy(pos_hbm.at[0, pl.ds(tok0, CH)], i0_v.at[p])
            pltpu.sync_copy(pos_hbm.at[1, pl.ds(tok0, CH)], i1_v.at[p])
            pltpu.sync_copy(wgt_hbm.at[0, pl.ds(tok0, CH)], w0_v.at[p])
            pltpu.sync_copy(wgt_hbm.at[1, pl.ds(tok0, CH)], w1_v.at[p])
            return (pltpu.async_copy(ys_hbm.at[i0_v.at[p]], a_v.at[p],
                                     sems[p][0]),
                    pltpu.async_copy(ys_hbm.at[i1_v.at[p]], b_v.at[p],
                                     sems[p][1]))

        pend = issue(0)
        for j in range(NCHUNK):
            p = j % 2
            nxt = issue(j + 1) if j + 1 < NCHUNK else None
            pend[0].wait()
            pend[1].wait()

            for r in range(CH):
                w0 = w0_v[p, r, :]
                w1 = w1_v[p, r, :]

                def col(cc, c3, _p=p, _r=r, _w0=w0, _w1=w1):
                    for u in range(4):
                        sl = pl.ds((cc * 4 + u) * LANES, LANES)
                        a_v[_p, _r, sl] = (_w0 * a_v[_p, _r, sl]
                                           + _w1 * b_v[_p, _r, sl])
                    return c3
                lax.fori_loop(0, D // (4 * LANES), col, 0)

            tok0 = wid * TPW + j * CH
            pltpu.sync_copy(a_v.at[p], out_hbm.at[pl.ds(tok0, CH)])
            pend = nxt

    return run(ys, pos, wgt)


def kernel(x, gate_w, gate_b, w1, b1, w2, b2):
    B, L, D = x.shape
    N = B * L
    E = gate_w.shape[1]
    T = (N * TOPK) // TILE + (E - 1)
    P = T * TILE

    xf = x.reshape(N, D)
    pos, wgt, tile_eid = _gating(xf, gate_w, gate_b, T)
    tile_eid = tile_eid.reshape(T)
    xs = _dispatch(xf, pos, P)
    h = _k1(tile_eid, xs, w1, b1, T)
    ys = _k2(tile_eid, h, w2, b2, T)
    out = _combine(ys, pos, wgt, N, D)
    return out.reshape(B, L, D)


# speedup vs baseline: 1.3915x; 1.0408x over previous
import functools

import jax
import jax.numpy as jnp
from jax import lax
from jax.experimental import pallas as pl
from jax.experimental.pallas import tpu as pltpu
import jax.experimental.pallas.tpu_sc as plsc

TOPK = 2
TILE = 128

NC, NS, LANES = 2, 16, 16
NW = NC * NS


def _gelu(v):
    return 0.5 * v * (1.0 + lax.erf(v * 0.7071067811865476))


def _cumsum0(v):
    n = v.shape[0]
    k = 1
    while k < n:
        shifted = jnp.concatenate(
            [jnp.zeros((k, v.shape[1]), v.dtype), v[:-k]], axis=0)
        v = v + shifted
        k *= 2
    return v


def _gating_body(T, E, x_ref, gw_ref, gb_ref, pos_ref, wgt_ref, eid_ref):
    N = x_ref.shape[0]
    x = x_ref[...]
    logits = jnp.dot(x, gw_ref[...], preferred_element_type=jnp.float32)
    logits = logits + gb_ref[...]
    iota_e = lax.broadcasted_iota(jnp.int32, (N, E), 1)

    m1 = jnp.max(logits, axis=1, keepdims=True)
    i1 = jnp.min(jnp.where(logits == m1, iota_e, E), axis=1, keepdims=True)
    l2 = jnp.where(iota_e == i1, -jnp.inf, logits)
    m2 = jnp.max(l2, axis=1, keepdims=True)
    i2 = jnp.min(jnp.where(l2 == m2, iota_e, E), axis=1, keepdims=True)

    b = jnp.exp(m2 - m1)
    wa = 1.0 / (1.0 + b)
    wb = b / (1.0 + b)

    oh1 = (iota_e == i1).astype(jnp.float32)
    oh2 = (iota_e == i2).astype(jnp.float32)
    both = oh1 + oh2
    S = _cumsum0(both) - both
    counts = jnp.sum(both, axis=0, keepdims=True)
    pc = jnp.ceil(counts * (1.0 / TILE)) * TILE
    tri = (lax.broadcasted_iota(jnp.int32, (E, E), 0)
           < lax.broadcasted_iota(jnp.int32, (E, E), 1)).astype(jnp.float32)
    base = jnp.dot(pc, tri)

    pos1 = jnp.sum((base + S) * oh1, axis=1)
    pos2 = jnp.sum((base + S) * oh2, axis=1)
    pos_ref[0, :] = pos1.astype(jnp.int32)
    pos_ref[1, :] = pos2.astype(jnp.int32)
    wgt_ref[0, :, :] = jnp.broadcast_to(wa, (N, LANES))
    wgt_ref[1, :, :] = jnp.broadcast_to(wb, (N, LANES))

    ts = lax.broadcasted_iota(jnp.int32, (T, E), 0).astype(jnp.float32) \
        * float(TILE)
    cnt = jnp.sum((ts >= base).astype(jnp.int32), axis=1)
    eid_ref[0, :] = jnp.minimum(cnt - 1, E - 1)


def _gating(xf, gate_w, gate_b, T):
    N, D = xf.shape
    E = gate_w.shape[1]
    return pl.pallas_call(
        functools.partial(_gating_body, T, E),
        out_shape=(
            jax.ShapeDtypeStruct((TOPK, N), jnp.int32),
            jax.ShapeDtypeStruct((TOPK, N, LANES), jnp.float32),
            jax.ShapeDtypeStruct((1, T), jnp.int32),
        ),
    )(xf, gate_w, gate_b.reshape(1, E))


def _dispatch(xf, pos, P):
    N, D = xf.shape
    TPW = N // NW
    mesh = plsc.VectorSubcoreMesh(core_axis_name="c", subcore_axis_name="s")

    @functools.partial(
        pl.kernel,
        out_type=jax.ShapeDtypeStruct((P, D), jnp.float32),
        mesh=mesh,
        scratch_types=[
            pltpu.VMEM((TPW, D), jnp.float32),
            pltpu.VMEM((TPW,), jnp.int32),
            pltpu.VMEM((TPW,), jnp.int32),
            pltpu.SemaphoreType.DMA,
        ],
    )
    def run(x_hbm, pos_hbm, xs_hbm, rows_v, idx0_v, idx1_v, sem):
        wid = lax.axis_index("s") * NC + lax.axis_index("c")
        tok0 = wid * TPW
        pltpu.sync_copy(x_hbm.at[pl.ds(tok0, TPW)], rows_v)
        pltpu.sync_copy(pos_hbm.at[0, pl.ds(tok0, TPW)], idx0_v)
        pltpu.sync_copy(pos_hbm.at[1, pl.ds(tok0, TPW)], idx1_v)
        pltpu.async_copy(rows_v, xs_hbm.at[idx0_v], sem).wait()
        pltpu.async_copy(rows_v, xs_hbm.at[idx1_v], sem).wait()

    return run(xf, pos)


def _run_info(tile_eid, E):
    change = jnp.concatenate(
        [jnp.zeros((1,), jnp.int32),
         (tile_eid[1:] != tile_eid[:-1]).astype(jnp.int32)])
    run_id = jnp.cumsum(change)
    nruns = (run_id[-1] + 1).reshape(1)
    run_expert = jnp.zeros((E,), jnp.int32).at[run_id].set(tile_eid)
    return run_id, run_expert, nruns


def _gmm_body(out_fn, cast_lhs, eid_ref, rid_ref, rexp_ref, nr_ref,
              lhs_ref, w_any, b_ref, o_ref, stag_ref, wbf_ref, s0, s1):
    t = pl.program_id(0)
    T = pl.num_programs(0)
    r = rid_ref[t]
    R = nr_ref[0]
    E = rexp_ref.shape[0]
    sems = (s0, s1)

    def prepare(q):
        for p in (0, 1):
            @pl.when(q % 2 == p)
            def _(p=p):
                pltpu.make_async_copy(
                    w_any.at[rexp_ref[jnp.minimum(q, E - 1)]],
                    stag_ref.at[p], sems[p]).wait()
                wbf_ref[p] = stag_ref[p].astype(jnp.bfloat16)

                @pl.when(q + 2 < R)
                def _():
                    pltpu.make_async_copy(
                        w_any.at[rexp_ref[jnp.minimum(q + 2, E - 1)]],
                        stag_ref.at[p], sems[p]).start()

    @pl.when(t == 0)
    def _():
        pltpu.make_async_copy(
            w_any.at[rexp_ref[0]], stag_ref.at[0], s0).start()

        @pl.when(R > 1)
        def _():
            pltpu.make_async_copy(
                w_any.at[rexp_ref[1]], stag_ref.at[1], s1).start()
        prepare(0)

    nxt_start = (t + 1 < T) & (rid_ref[jnp.minimum(t + 1, T - 1)] != r)

    @pl.when(nxt_start)
    def _():
        prepare(r + 1)

    lhs = lhs_ref[...]
    if cast_lhs:
        lhs = lhs.astype(jnp.bfloat16)
    acc = jnp.dot(lhs, wbf_ref[r % 2], preferred_element_type=jnp.float32)
    o_ref[...] = out_fn(acc + b_ref[0])


def _gmm(tile_eid, lhs, w, b, T, out_dtype, out_fn, cast_lhs):
    P = lhs.shape[0]
    E, K, M = w.shape
    run_id, run_expert, nruns = _run_info(tile_eid, E)
    grid_spec = pltpu.PrefetchScalarGridSpec(
        num_scalar_prefetch=4,
        grid=(T,),
        in_specs=[
            pl.BlockSpec((TILE, K), lambda t, eid, rid, rexp, nr: (t, 0)),
            pl.BlockSpec(memory_space=pl.ANY),
            pl.BlockSpec((1, 1, M),
                         lambda t, eid, rid, rexp, nr: (eid[t], 0, 0)),
        ],
        out_specs=pl.BlockSpec((TILE, M), lambda t, eid, rid, rexp, nr:
                               (t, 0)),
        scratch_shapes=[
            pltpu.VMEM((2, K, M), jnp.float32),
            pltpu.VMEM((2, K, M), jnp.bfloat16),
            pltpu.SemaphoreType.DMA,
            pltpu.SemaphoreType.DMA,
        ],
    )
    return pl.pallas_call(
        functools.partial(_gmm_body, out_fn, cast_lhs),
        grid_spec=grid_spec,
        out_shape=jax.ShapeDtypeStruct((P, M), out_dtype),
        compiler_params=pltpu.CompilerParams(
            dimension_semantics=("arbitrary",)),
    )(tile_eid, run_id, run_expert, nruns, lhs, w, b.reshape(E, 1, M))


def _k1(tile_eid, xs, w1, b1, T):
    return _gmm(tile_eid, xs, w1, b1, T, jnp.bfloat16,
                lambda v: _gelu(v).astype(jnp.bfloat16), cast_lhs=True)


def _k2(tile_eid, h, w2, b2, T):
    return _gmm(tile_eid, h, w2, b2, T, jnp.float32,
                lambda v: v, cast_lhs=False)


def _combine(ys, pos, wgt, N, D):
    TPW = N // NW
    CH = 16
    NCHUNK = TPW // CH
    mesh = plsc.VectorSubcoreMesh(core_axis_name="c", subcore_axis_name="s")

    @functools.partial(
        pl.kernel,
        out_type=jax.ShapeDtypeStruct((N, D), jnp.float32),
        mesh=mesh,
        scratch_types=[
            pltpu.VMEM((2, CH, D), jnp.float32),
            pltpu.VMEM((2, CH, D), jnp.float32),
            pltpu.VMEM((2, CH), jnp.int32),
            pltpu.VMEM((2, CH), jnp.int32),
            pltpu.VMEM((2, CH, LANES), jnp.float32),
            pltpu.VMEM((2, CH, LANES), jnp.float32),
            pltpu.SemaphoreType.DMA,
            pltpu.SemaphoreType.DMA,
            pltpu.SemaphoreType.DMA,
            pltpu.SemaphoreType.DMA,
        ],
    )
    def run(ys_hbm, pos_hbm, wgt_hbm, out_hbm,
            a_v, b_v, i0_v, i1_v, w0_v, w1_v, s_a0, s_b0, s_a1, s_b1):
        wid = lax.axis_index("s") * NC + lax.axis_index("c")
        sems = ((s_a0, s_b0), (s_a1, s_b1))

        def issue(j):
            p = j % 2
            tok0 = wid * TPW + j * CH
            pltpu.sync_copy(pos_hbm.at[0, pl.ds(tok0, CH)], i0_v.at[p])
            pltpu.sync_copy(pos_hbm.at[1, pl.ds(tok0, CH)], i1_v.at[p])
            pltpu.sync_copy(wgt_hbm.at[0, pl.ds(tok0, CH)], w0_v.at[p])
            pltpu.sync_copy(wgt_hbm.at[1, pl.ds(tok0, CH)], w1_v.at[p])
            return (pltpu.async_copy(ys_hbm.at[i0_v.at[p]], a_v.at[p],
                                     sems[p][0]),
                    pltpu.async_copy(ys_hbm.at[i1_v.at[p]], b_v.at[p],
                                     sems[p][1]))

        pend = issue(0)
        for j in range(NCHUNK):
            p = j % 2
            nxt = issue(j + 1) if j + 1 < NCHUNK else None
            pend[0].wait()
            pend[1].wait()

            for r in range(CH):
                w0 = w0_v[p, r, :]
                w1 = w1_v[p, r, :]

                def col(cc, c3, _p=p, _r=r, _w0=w0, _w1=w1):
                    for u in range(8):
                        sl = pl.ds((cc * 8 + u) * LANES, LANES)
                        a_v[_p, _r, sl] = (_w0 * a_v[_p, _r, sl]
                                           + _w1 * b_v[_p, _r, sl])
                    return c3
                lax.fori_loop(0, D // (8 * LANES), col, 0)

            tok0 = wid * TPW + j * CH
            pltpu.sync_copy(a_v.at[p], out_hbm.at[pl.ds(tok0, CH)])
            pend = nxt

    return run(ys, pos, wgt)


def kernel(x, gate_w, gate_b, w1, b1, w2, b2):
    B, L, D = x.shape
    N = B * L
    E = gate_w.shape[1]
    T = (N * TOPK) // TILE + (E - 1)
    P = T * TILE

    xf = x.reshape(N, D)
    pos, wgt, tile_eid = _gating(xf, gate_w, gate_b, T)
    tile_eid = tile_eid.reshape(T)
    xs = _dispatch(xf, pos, P)
    h = _k1(tile_eid, xs, w1, b1, T)
    ys = _k2(tile_eid, h, w2, b2, T)
    out = _combine(ys, pos, wgt, N, D)
    return out.reshape(B, L, D)
